# Initial kernel scaffold; baseline (speedup 1.0000x reference)
import jax
import jax.numpy as jnp
from jax.experimental import pallas as pl


def kernel(x, edge_index, batch, params):
    raise NotImplementedError("write your pallas kernel here")


# speedup vs baseline: 1.0053x; 1.0053x over previous
import jax
import jax.numpy as jnp
from jax.experimental import pallas as pl

_H = 6
_D = 128


def _proj_body(z_ref, wl_ref, bl_ref, wr_ref, br_ref, xl_ref, xr_ref):
    z = z_ref[...]
    xl_ref[...] = jnp.dot(z, wl_ref[...], preferred_element_type=jnp.float32) + bl_ref[...]
    xr_ref[...] = jnp.dot(z, wr_ref[...], preferred_element_type=jnp.float32) + br_ref[...]


def _proj(z, WlT, bl, WrT, br):
    n, cin = z.shape
    cout = WlT.shape[1]
    blk = 400
    grid = n // blk
    return pl.pallas_call(
        _proj_body,
        grid=(grid,),
        in_specs=[
            pl.BlockSpec((blk, cin), lambda i: (i, 0)),
            pl.BlockSpec((cin, cout), lambda i: (0, 0)),
            pl.BlockSpec((1, cout), lambda i: (0, 0)),
            pl.BlockSpec((cin, cout), lambda i: (0, 0)),
            pl.BlockSpec((1, cout), lambda i: (0, 0)),
        ],
        out_specs=[
            pl.BlockSpec((blk, cout), lambda i: (i, 0)),
            pl.BlockSpec((blk, cout), lambda i: (i, 0)),
        ],
        out_shape=[jax.ShapeDtypeStruct((n, cout), jnp.float32)] * 2,
    )(z, WlT, bl.reshape(1, -1), WrT, br.reshape(1, -1))


def _post_body(y_ref, bnw_ref, bnb_ref, res_ref, o_ref):
    y = y_ref[...]
    m = jnp.mean(y, axis=0, keepdims=True)
    v = jnp.mean((y - m) * (y - m), axis=0, keepdims=True)
    z = (y - m) * jax.lax.rsqrt(v + 1e-5) * bnw_ref[...] + bnb_ref[...]
    z = jnp.maximum(z, 0.0)
    mm = jnp.mean(z, axis=-1, keepdims=True)
    vv = jnp.mean((z - mm) * (z - mm), axis=-1, keepdims=True)
    o_ref[...] = (z - mm) * jax.lax.rsqrt(vv + 1e-5) + res_ref[...]


def _post(y, bnw, bnb, res):
    n, d = y.shape
    return pl.pallas_call(
        _post_body,
        in_specs=[
            pl.BlockSpec((n, d), lambda: (0, 0)),
            pl.BlockSpec((1, d), lambda: (0, 0)),
            pl.BlockSpec((1, d), lambda: (0, 0)),
            pl.BlockSpec((n, d), lambda: (0, 0)),
        ],
        out_specs=pl.BlockSpec((n, d), lambda: (0, 0)),
        out_shape=jax.ShapeDtypeStruct((n, d), jnp.float32),
    )(y, bnw.reshape(1, -1), bnb.reshape(1, -1), res)


def _bn_rows(z, w, b):
    m = jnp.mean(z, axis=0, keepdims=True)
    v = jnp.mean((z - m) * (z - m), axis=0, keepdims=True)
    return (z - m) * jax.lax.rsqrt(v + 1e-5) * w + b


def _s2s_body(h_ref, bm_ref, wih_ref, whh_ref, bih_ref, bhh_ref,
              wrep_ref, brep_ref, bnwrep_ref, bnbrep_ref,
              h1_ref, h2_ref, o1_ref, o2_ref):
    h = h_ref[...]
    nb = o1_ref.shape[0]
    bcol = bm_ref[...]
    iot = jax.lax.broadcasted_iota(jnp.int32, (h.shape[0], nb), 1)
    M = (bcol == iot).astype(jnp.float32)

    wih = wih_ref[...]
    whh = whh_ref[...]
    bih = bih_ref[...]
    bhh = bhh_ref[...]

    d = h.shape[1]
    qs = jnp.zeros((nb, 2 * d), jnp.float32)
    hh = jnp.zeros((nb, d), jnp.float32)
    cc = jnp.zeros((nb, d), jnp.float32)
    for _ in range(3):
        gates = (jnp.dot(qs, wih, preferred_element_type=jnp.float32) + bih
                 + jnp.dot(hh, whh, preferred_element_type=jnp.float32) + bhh)
        gi = gates[:, 0 * d:1 * d]
        gf = gates[:, 1 * d:2 * d]
        gg = gates[:, 2 * d:3 * d]
        go = gates[:, 3 * d:4 * d]
        cc = jax.nn.sigmoid(gf) * cc + jax.nn.sigmoid(gi) * jnp.tanh(gg)
        hh = jax.nn.sigmoid(go) * jnp.tanh(cc)
        q = hh
        qb = jnp.dot(M, q, preferred_element_type=jnp.float32)
        e = jnp.sum(h * qb, axis=-1, keepdims=True)
        we = jnp.where(M > 0, e, -1e30)
        em = jnp.max(we, axis=0, keepdims=True)
        emn = jnp.dot(M, em.T, preferred_element_type=jnp.float32)
        ex = jnp.exp(e - emn)
        den = jax.lax.dot_general(M, ex, (((0,), (0,)), ((), ())),
                                  preferred_element_type=jnp.float32)
        a = ex / (jnp.dot(M, den, preferred_element_type=jnp.float32) + 1e-16)
        r = jax.lax.dot_general(M, a * h, (((0,), (0,)), ((), ())),
                                preferred_element_type=jnp.float32)
        qs = jnp.concatenate([q, r], axis=-1)

    rep = jnp.maximum(jnp.dot(qs, wrep_ref[...], preferred_element_type=jnp.float32)
                      + brep_ref[...], 0.0)
    rep = _bn_rows(rep, bnwrep_ref[...], bnbrep_ref[...])

    for hp_ref, o_ref in ((h1_ref, o1_ref), (h2_ref, o2_ref)):
        hp = hp_ref[...]
        w1 = hp[0:128, 0:64]
        b1 = hp[128:129, 0:64]
        bw1 = hp[129:130, 0:64]
        bb1 = hp[130:131, 0:64]
        w2 = hp[131:195, 0:32]
        b2 = hp[195:196, 0:32]
        bw2 = hp[196:197, 0:32]
        bb2 = hp[197:198, 0:32]
        w3 = hp[198:230, 0:1]
        b3 = hp[230:231, 0:1]
        z1 = jnp.maximum(jnp.dot(rep, w1, preferred_element_type=jnp.float32) + b1, 0.0)
        z1 = _bn_rows(z1, bw1, bb1)
        z2 = jnp.maximum(jnp.dot(z1, w2, preferred_element_type=jnp.float32) + b2, 0.0)
        z2 = _bn_rows(z2, bw2, bb2)
        o_ref[...] = jnp.dot(z2, w3, preferred_element_type=jnp.float32) + b3


def _pack_head(hp):
    p = jnp.zeros((232, 128), jnp.float32)
    p = p.at[0:128, 0:64].set(hp[0].T)
    p = p.at[128, 0:64].set(hp[1])
    p = p.at[129, 0:64].set(hp[2])
    p = p.at[130, 0:64].set(hp[3])
    p = p.at[131:195, 0:32].set(hp[4].T)
    p = p.at[195, 0:32].set(hp[5])
    p = p.at[196, 0:32].set(hp[6])
    p = p.at[197, 0:32].set(hp[7])
    p = p.at[198:230, 0:1].set(hp[8].T)
    p = p.at[230, 0:1].set(hp[9])
    return p


def _s2s(h, batch, lstm, rep, gcgr, glp1r):
    n, d = h.shape
    nb = 64
    Wih, Whh, bih, bhh = lstm
    Wr_, br_, bnw, bnb = rep
    args = (h, batch.astype(jnp.int32).reshape(n, 1),
            Wih.T, Whh.T, bih.reshape(1, -1), bhh.reshape(1, -1),
            Wr_.T, br_.reshape(1, -1), bnw.reshape(1, -1), bnb.reshape(1, -1),
            _pack_head(gcgr), _pack_head(glp1r))
    full = lambda a: pl.BlockSpec(a.shape, lambda: (0,) * a.ndim)
    return pl.pallas_call(
        _s2s_body,
        in_specs=[full(a) for a in args],
        out_specs=[pl.BlockSpec((nb, 1), lambda: (0, 0))] * 2,
        out_shape=[jax.ShapeDtypeStruct((nb, 1), jnp.float32)] * 2,
    )(*args)


def kernel(x, edge_index, batch, params):
    n = x.shape[0]
    loops = jnp.arange(n, dtype=edge_index.dtype)
    src = jnp.concatenate([edge_index[0], loops])
    dst = jnp.concatenate([edge_index[1], loops])

    h = x
    res = None
    for i, cp in enumerate(params["convs"]):
        Wl, bl, Wr, br, att, bias, bnw, bnb = cp
        if i > 0:
            res = h
        xl, xr = _proj(h, Wl.T, bl, Wr.T, br)
        xl3 = xl.reshape(n, _H, _D)
        xr3 = xr.reshape(n, _H, _D)
        e = jnp.sum(jax.nn.leaky_relu(xl3[src] + xr3[dst], 0.2) * att, axis=-1)
        emax = jax.ops.segment_max(e, dst, num_segments=n)
        ex = jnp.exp(e - emax[dst])
        den = jax.ops.segment_sum(ex, dst, num_segments=n)
        alpha = ex / (den[dst] + 1e-16)
        out = jax.ops.segment_sum(alpha[:, :, None] * xl3[src], dst, num_segments=n)
        y = out.mean(axis=1) + bias
        rz = res if i == 1 else jnp.zeros_like(y)
        h = _post(y, bnw, bnb, rz)

    return _s2s(h, batch, params["lstm"], params["rep"],
                params["gcgr"], params["glp1r"])


# speedup vs baseline: 9.9188x; 9.8662x over previous
import functools

import jax
import jax.numpy as jnp
from jax import lax
from jax.experimental import pallas as pl
from jax.experimental.pallas import tpu as pltpu
from jax.experimental.pallas import tpu_sc as plsc

_H = 6
_D = 128
_NW = 32
_C = 80
_TP = 10240


def _sc_gather(table, idx3):
    nch = idx3.shape[1]
    W = table.shape[1]
    per_w = nch * _C
    ep = _NW * per_w
    mesh = plsc.VectorSubcoreMesh(core_axis_name="c", subcore_axis_name="s")

    @functools.partial(
        pl.kernel, mesh=mesh,
        out_type=jax.ShapeDtypeStruct((ep, W), jnp.float32),
        scratch_types=[
            pltpu.VMEM((_C,), jnp.int32),
            pltpu.VMEM((_C, W), jnp.float32),
            pltpu.SemaphoreType.DMA,
        ],
    )
    def k(table_hbm, idx_hbm, out_hbm, idxv, buf, sem):
        wid = lax.axis_index("s") * 2 + lax.axis_index("c")

        def body(c, carry):
            pltpu.sync_copy(idx_hbm.at[wid, c], idxv)
            pltpu.async_copy(table_hbm.at[idxv], buf, sem).wait()
            pltpu.sync_copy(buf, out_hbm.at[pl.ds(wid * per_w + c * _C, _C)])
            return carry

        lax.fori_loop(0, nch, body, 0)

    return k(table, idx3)


def _sc_scatter_add(rows, idx3, zer):
    nch = idx3.shape[1]
    W = rows.shape[1]
    per_w = nch * _C
    stripe = _TP // 16
    mesh = plsc.VectorSubcoreMesh(core_axis_name="c", subcore_axis_name="s")

    @functools.partial(
        pl.kernel, mesh=mesh,
        out_type=jax.ShapeDtypeStruct((2, _TP, W), jnp.float32),
        scratch_types=[
            pltpu.VMEM((_C,), jnp.int32),
            pltpu.VMEM((_C, W), jnp.float32),
            pltpu.VMEM((8, W), jnp.float32),
            pltpu.VMEM_SHARED((_TP, W), jnp.float32),
            pltpu.SemaphoreType.DMA,
        ],
    )
    def k(rows_hbm, idx_hbm, zer_hbm, out_hbm, idxv, rowv, zbuf, acc, sem):
        cid = lax.axis_index("c")
        sid = lax.axis_index("s")
        wid = sid * 2 + cid

        pltpu.sync_copy(zer_hbm, zbuf)

        def zbody(t, carry):
            pltpu.sync_copy(zbuf, acc.at[pl.ds(sid * stripe + t * 8, 8)])
            return carry

        lax.fori_loop(0, stripe // 8, zbody, 0)
        plsc.subcore_barrier()

        def body(c, carry):
            pltpu.sync_copy(idx_hbm.at[wid, c], idxv)
            pltpu.sync_copy(rows_hbm.at[pl.ds(wid * per_w + c * _C, _C)], rowv)
            pltpu.sync_copy(rowv, acc.at[idxv], add=True)
            return carry

        lax.fori_loop(0, nch, body, 0)
        plsc.subcore_barrier()
        pltpu.sync_copy(acc.at[pl.ds(sid * stripe, stripe)],
                        out_hbm.at[cid, pl.ds(sid * stripe, stripe)])

    return k(rows, idx3, zer)


def _logits_body(a_ref, b_ref, att_ref, p_ref):
    s = a_ref[...] + b_ref[...]
    s = jnp.where(s >= 0, s, 0.2 * s)
    att = att_ref[...]
    cols = []
    for h in range(_H):
        cols.append(jnp.sum(s[:, h * _D:(h + 1) * _D] * att[h:h + 1, :],
                            axis=-1, keepdims=True))
    cols.append(jnp.zeros((s.shape[0], _D - _H), jnp.float32))
    p_ref[...] = jnp.exp(jnp.concatenate(cols, axis=-1))


def _tc_logits(xls, xrd, attp):
    ep = xls.shape[0]
    blk = 512
    return pl.pallas_call(
        _logits_body,
        grid=(ep // blk,),
        in_specs=[
            pl.BlockSpec((blk, _H * _D), lambda i: (i, 0)),
            pl.BlockSpec((blk, _H * _D), lambda i: (i, 0)),
            pl.BlockSpec((8, _D), lambda i: (0, 0)),
        ],
        out_specs=pl.BlockSpec((blk, _D), lambda i: (i, 0)),
        out_shape=jax.ShapeDtypeStruct((ep, _D), jnp.float32),
    )(xls, xrd, attp)


def _contrib_body(x_ref, p_ref, d0_ref, d1_ref, w_ref):
    x = x_ref[...]
    p = p_ref[...]
    den = d0_ref[...] + d1_ref[...]
    acc = jnp.zeros((x.shape[0], _D), jnp.float32)
    for h in range(_H):
        a = p[:, h:h + 1] / (den[:, h:h + 1] + 1e-16)
        acc = acc + a * x[:, h * _D:(h + 1) * _D]
    w_ref[...] = acc * (1.0 / _H)


def _tc_contrib(xls, p, d0e, d1e):
    ep = xls.shape[0]
    blk = 512
    return pl.pallas_call(
        _contrib_body,
        grid=(ep // blk,),
        in_specs=[
            pl.BlockSpec((blk, _H * _D), lambda i: (i, 0)),
            pl.BlockSpec((blk, _D), lambda i: (i, 0)),
            pl.BlockSpec((blk, _D), lambda i: (i, 0)),
            pl.BlockSpec((blk, _D), lambda i: (i, 0)),
        ],
        out_specs=pl.BlockSpec((blk, _D), lambda i: (i, 0)),
        out_shape=jax.ShapeDtypeStruct((ep, _D), jnp.float32),
    )(xls, p, d0e, d1e)


def _proj_body(z_ref, wl_ref, bl_ref, wr_ref, br_ref, xl_ref, xr_ref):
    z = z_ref[...]
    xl_ref[...] = jnp.dot(z, wl_ref[...], preferred_element_type=jnp.float32) + bl_ref[...]
    xr_ref[...] = jnp.dot(z, wr_ref[...], preferred_element_type=jnp.float32) + br_ref[...]


def _proj(z, WlT, bl, WrT, br):
    n, cin = z.shape
    cout = WlT.shape[1]
    blk = 400
    grid = n // blk
    return pl.pallas_call(
        _proj_body,
        grid=(grid,),
        in_specs=[
            pl.BlockSpec((blk, cin), lambda i: (i, 0)),
            pl.BlockSpec((cin, cout), lambda i: (0, 0)),
            pl.BlockSpec((1, cout), lambda i: (0, 0)),
            pl.BlockSpec((cin, cout), lambda i: (0, 0)),
            pl.BlockSpec((1, cout), lambda i: (0, 0)),
        ],
        out_specs=[
            pl.BlockSpec((blk, cout), lambda i: (i, 0)),
            pl.BlockSpec((blk, cout), lambda i: (i, 0)),
        ],
        out_shape=[jax.ShapeDtypeStruct((n, cout), jnp.float32)] * 2,
    )(z, WlT, bl.reshape(1, -1), WrT, br.reshape(1, -1))


def _post_body(y0_ref, y1_ref, bias_ref, bnw_ref, bnb_ref, res_ref, o_ref):
    y = y0_ref[...] + y1_ref[...] + bias_ref[...]
    m = jnp.mean(y, axis=0, keepdims=True)
    v = jnp.mean((y - m) * (y - m), axis=0, keepdims=True)
    z = (y - m) * jax.lax.rsqrt(v + 1e-5) * bnw_ref[...] + bnb_ref[...]
    z = jnp.maximum(z, 0.0)
    mm = jnp.mean(z, axis=-1, keepdims=True)
    vv = jnp.mean((z - mm) * (z - mm), axis=-1, keepdims=True)
    o_ref[...] = (z - mm) * jax.lax.rsqrt(vv + 1e-5) + res_ref[...]


def _post(y0, y1, bias, bnw, bnb, res):
    n, d = y0.shape
    return pl.pallas_call(
        _post_body,
        in_specs=[
            pl.BlockSpec((n, d), lambda: (0, 0)),
            pl.BlockSpec((n, d), lambda: (0, 0)),
            pl.BlockSpec((1, d), lambda: (0, 0)),
            pl.BlockSpec((1, d), lambda: (0, 0)),
            pl.BlockSpec((1, d), lambda: (0, 0)),
            pl.BlockSpec((n, d), lambda: (0, 0)),
        ],
        out_specs=pl.BlockSpec((n, d), lambda: (0, 0)),
        out_shape=jax.ShapeDtypeStruct((n, d), jnp.float32),
    )(y0, y1, bias.reshape(1, -1), bnw.reshape(1, -1), bnb.reshape(1, -1), res)


def _bn_rows(z, w, b):
    m = jnp.mean(z, axis=0, keepdims=True)
    v = jnp.mean((z - m) * (z - m), axis=0, keepdims=True)
    return (z - m) * jax.lax.rsqrt(v + 1e-5) * w + b


def _s2s_body(h_ref, bm_ref, wih_ref, whh_ref, bih_ref, bhh_ref,
              wrep_ref, brep_ref, bnwrep_ref, bnbrep_ref,
              h1_ref, h2_ref, o1_ref, o2_ref):
    h = h_ref[...]
    nb = o1_ref.shape[0]
    bcol = bm_ref[...]
    iot = jax.lax.broadcasted_iota(jnp.int32, (h.shape[0], nb), 1)
    M = (bcol == iot).astype(jnp.float32)

    wih = wih_ref[...]
    whh = whh_ref[...]
    bih = bih_ref[...]
    bhh = bhh_ref[...]

    d = h.shape[1]
    qs = jnp.zeros((nb, 2 * d), jnp.float32)
    hh = jnp.zeros((nb, d), jnp.float32)
    cc = jnp.zeros((nb, d), jnp.float32)
    for _ in range(3):
        gates = (jnp.dot(qs, wih, preferred_element_type=jnp.float32) + bih
                 + jnp.dot(hh, whh, preferred_element_type=jnp.float32) + bhh)
        gi = gates[:, 0 * d:1 * d]
        gf = gates[:, 1 * d:2 * d]
        gg = gates[:, 2 * d:3 * d]
        go = gates[:, 3 * d:4 * d]
        cc = jax.nn.sigmoid(gf) * cc + jax.nn.sigmoid(gi) * jnp.tanh(gg)
        hh = jax.nn.sigmoid(go) * jnp.tanh(cc)
        q = hh
        qb = jnp.dot(M, q, preferred_element_type=jnp.float32)
        e = jnp.sum(h * qb, axis=-1, keepdims=True)
        we = jnp.where(M > 0, e, -1e30)
        em = jnp.max(we, axis=0, keepdims=True)
        emn = jnp.dot(M, em.T, preferred_element_type=jnp.float32)
        ex = jnp.exp(e - emn)
        den = jax.lax.dot_general(M, ex, (((0,), (0,)), ((), ())),
                                  preferred_element_type=jnp.float32)
        a = ex / (jnp.dot(M, den, preferred_element_type=jnp.float32) + 1e-16)
        r = jax.lax.dot_general(M, a * h, (((0,), (0,)), ((), ())),
                                preferred_element_type=jnp.float32)
        qs = jnp.concatenate([q, r], axis=-1)

    rep = jnp.maximum(jnp.dot(qs, wrep_ref[...], preferred_element_type=jnp.float32)
                      + brep_ref[...], 0.0)
    rep = _bn_rows(rep, bnwrep_ref[...], bnbrep_ref[...])

    for hp_ref, o_ref in ((h1_ref, o1_ref), (h2_ref, o2_ref)):
        hp = hp_ref[...]
        w1 = hp[0:128, 0:64]
        b1 = hp[128:129, 0:64]
        bw1 = hp[129:130, 0:64]
        bb1 = hp[130:131, 0:64]
        w2 = hp[131:195, 0:32]
        b2 = hp[195:196, 0:32]
        bw2 = hp[196:197, 0:32]
        bb2 = hp[197:198, 0:32]
        w3 = hp[198:230, 0:1]
        b3 = hp[230:231, 0:1]
        z1 = jnp.maximum(jnp.dot(rep, w1, preferred_element_type=jnp.float32) + b1, 0.0)
        z1 = _bn_rows(z1, bw1, bb1)
        z2 = jnp.maximum(jnp.dot(z1, w2, preferred_element_type=jnp.float32) + b2, 0.0)
        z2 = _bn_rows(z2, bw2, bb2)
        o_ref[...] = jnp.dot(z2, w3, preferred_element_type=jnp.float32) + b3


def _pack_head(hp):
    p = jnp.zeros((232, 128), jnp.float32)
    p = p.at[0:128, 0:64].set(hp[0].T)
    p = p.at[128, 0:64].set(hp[1])
    p = p.at[129, 0:64].set(hp[2])
    p = p.at[130, 0:64].set(hp[3])
    p = p.at[131:195, 0:32].set(hp[4].T)
    p = p.at[195, 0:32].set(hp[5])
    p = p.at[196, 0:32].set(hp[6])
    p = p.at[197, 0:32].set(hp[7])
    p = p.at[198:230, 0:1].set(hp[8].T)
    p = p.at[230, 0:1].set(hp[9])
    return p


def _s2s(h, batch, lstm, rep, gcgr, glp1r):
    n, d = h.shape
    nb = 64
    Wih, Whh, bih, bhh = lstm
    Wr_, br_, bnw, bnb = rep
    args = (h, batch.astype(jnp.int32).reshape(n, 1),
            Wih.T, Whh.T, bih.reshape(1, -1), bhh.reshape(1, -1),
            Wr_.T, br_.reshape(1, -1), bnw.reshape(1, -1), bnb.reshape(1, -1),
            _pack_head(gcgr), _pack_head(glp1r))
    full = lambda a: pl.BlockSpec(a.shape, lambda: (0,) * a.ndim)
    return pl.pallas_call(
        _s2s_body,
        in_specs=[full(a) for a in args],
        out_specs=[pl.BlockSpec((nb, 1), lambda: (0, 0))] * 2,
        out_shape=[jax.ShapeDtypeStruct((nb, 1), jnp.float32)] * 2,
    )(*args)


def kernel(x, edge_index, batch, params):
    n = x.shape[0]
    ne = edge_index.shape[1] + n
    ep = ((ne + _NW * _C - 1) // (_NW * _C)) * (_NW * _C)
    loops = jnp.arange(n, dtype=jnp.int32)
    padi = jnp.full((ep - ne,), n, jnp.int32)
    src3 = jnp.concatenate([edge_index[0].astype(jnp.int32), loops, padi]
                           ).reshape(_NW, -1, _C)
    dst3 = jnp.concatenate([edge_index[1].astype(jnp.int32), loops, padi]
                           ).reshape(_NW, -1, _C)
    zer128 = jnp.zeros((8, _D), jnp.float32)
    padrows = jnp.zeros((_TP - n, _H * _D), jnp.float32)

    h = x
    res = None
    for i, cp in enumerate(params["convs"]):
        Wl, bl, Wr, br, att, bias, bnw, bnb = cp
        if i > 0:
            res = h
        xl, xr = _proj(h, Wl.T, bl, Wr.T, br)
        xlp = jnp.concatenate([xl, padrows])
        xrp = jnp.concatenate([xr, padrows])
        attp = jnp.zeros((8, _D), jnp.float32).at[:_H].set(att)
        xls = _sc_gather(xlp, src3)
        xrd = _sc_gather(xrp, dst3)
        p = _tc_logits(xls, xrd, attp)
        denp = _sc_scatter_add(p, dst3, zer128)
        d0e = _sc_gather(denp[0], dst3)
        d1e = _sc_gather(denp[1], dst3)
        w = _tc_contrib(xls, p, d0e, d1e)
        outp = _sc_scatter_add(w, dst3, zer128)
        rz = res if i == 1 else jnp.zeros((n, _D), jnp.float32)
        h = _post(outp[0, :n], outp[1, :n], bias, bnw, bnb, rz)

    return _s2s(h, batch, params["lstm"], params["rep"],
                params["gcgr"], params["glp1r"])


# speedup vs baseline: 10.6363x; 1.0723x over previous
import functools

import jax
import jax.numpy as jnp
from jax import lax
from jax.experimental import pallas as pl
from jax.experimental.pallas import tpu as pltpu
from jax.experimental.pallas import tpu_sc as plsc

_H = 6
_D = 128
_NW = 32
_C = 80
_TP = 10240


def _sc_gather(table, idx3):
    nch = idx3.shape[1]
    W = table.shape[1]
    per_w = nch * _C
    ep = _NW * per_w
    mesh = plsc.VectorSubcoreMesh(core_axis_name="c", subcore_axis_name="s")

    @functools.partial(
        pl.kernel, mesh=mesh,
        out_type=jax.ShapeDtypeStruct((ep, W), jnp.float32),
        scratch_types=[
            pltpu.VMEM((_C,), jnp.int32),
            pltpu.VMEM((_C, W), jnp.float32),
            pltpu.SemaphoreType.DMA,
        ],
    )
    def k(table_hbm, idx_hbm, out_hbm, idxv, buf, sem):
        wid = lax.axis_index("s") * 2 + lax.axis_index("c")

        def body(c, carry):
            pltpu.sync_copy(idx_hbm.at[wid, c], idxv)
            pltpu.async_copy(table_hbm.at[idxv], buf, sem).wait()
            pltpu.sync_copy(buf, out_hbm.at[pl.ds(wid * per_w + c * _C, _C)])
            return carry

        lax.fori_loop(0, nch, body, 0)

    return k(table, idx3)


def _sc_scatter_add(rows, idx3, zer):
    nch = idx3.shape[1]
    W = rows.shape[1]
    per_w = nch * _C
    stripe = _TP // 16
    mesh = plsc.VectorSubcoreMesh(core_axis_name="c", subcore_axis_name="s")

    @functools.partial(
        pl.kernel, mesh=mesh,
        out_type=jax.ShapeDtypeStruct((2, _TP, W), jnp.float32),
        scratch_types=[
            pltpu.VMEM((_C,), jnp.int32),
            pltpu.VMEM((_C, W), jnp.float32),
            pltpu.VMEM((8, W), jnp.float32),
            pltpu.VMEM_SHARED((_TP, W), jnp.float32),
            pltpu.SemaphoreType.DMA,
        ],
    )
    def k(rows_hbm, idx_hbm, zer_hbm, out_hbm, idxv, rowv, zbuf, acc, sem):
        cid = lax.axis_index("c")
        sid = lax.axis_index("s")
        wid = sid * 2 + cid

        pltpu.sync_copy(zer_hbm, zbuf)

        def zbody(t, carry):
            pltpu.sync_copy(zbuf, acc.at[pl.ds(sid * stripe + t * 8, 8)])
            return carry

        lax.fori_loop(0, stripe // 8, zbody, 0)
        plsc.subcore_barrier()

        def body(c, carry):
            pltpu.sync_copy(idx_hbm.at[wid, c], idxv)
            pltpu.sync_copy(rows_hbm.at[pl.ds(wid * per_w + c * _C, _C)], rowv)
            pltpu.sync_copy(rowv, acc.at[idxv], add=True)
            return carry

        lax.fori_loop(0, nch, body, 0)
        plsc.subcore_barrier()
        pltpu.sync_copy(acc.at[pl.ds(sid * stripe, stripe)],
                        out_hbm.at[cid, pl.ds(sid * stripe, stripe)])

    return k(rows, idx3, zer)


def _logits_body(a_ref, b_ref, att_ref, p_ref):
    s = a_ref[...] + b_ref[...]
    s = jnp.where(s >= 0, s, 0.2 * s)
    att = att_ref[...]
    cols = []
    for h in range(_H):
        cols.append(jnp.sum(s[:, h * _D:(h + 1) * _D] * att[h:h + 1, :],
                            axis=-1, keepdims=True))
    cols.append(jnp.zeros((s.shape[0], _D - _H), jnp.float32))
    p_ref[...] = jnp.exp(jnp.concatenate(cols, axis=-1))


def _tc_logits(xls, xrd, attp):
    ep = xls.shape[0]
    blk = 512
    return pl.pallas_call(
        _logits_body,
        grid=(ep // blk,),
        in_specs=[
            pl.BlockSpec((blk, _H * _D), lambda i: (i, 0)),
            pl.BlockSpec((blk, _H * _D), lambda i: (i, 0)),
            pl.BlockSpec((8, _D), lambda i: (0, 0)),
        ],
        out_specs=pl.BlockSpec((blk, _D), lambda i: (i, 0)),
        out_shape=jax.ShapeDtypeStruct((ep, _D), jnp.float32),
    )(xls, xrd, attp)


def _add2_body(a_ref, b_ref, o_ref):
    o_ref[...] = a_ref[...] + b_ref[...]


def _tc_add2(a, b):
    blk = 1024
    return pl.pallas_call(
        _add2_body,
        grid=(a.shape[0] // blk,),
        in_specs=[pl.BlockSpec((blk, a.shape[1]), lambda i: (i, 0))] * 2,
        out_specs=pl.BlockSpec((blk, a.shape[1]), lambda i: (i, 0)),
        out_shape=jax.ShapeDtypeStruct(a.shape, jnp.float32),
    )(a, b)


def _contrib_body(x_ref, p_ref, d_ref, w_ref):
    x = x_ref[...]
    p = p_ref[...]
    den = d_ref[...]
    acc = jnp.zeros((x.shape[0], _D), jnp.float32)
    for h in range(_H):
        a = p[:, h:h + 1] / (den[:, h:h + 1] + 1e-16)
        acc = acc + a * x[:, h * _D:(h + 1) * _D]
    w_ref[...] = acc * (1.0 / _H)


def _tc_contrib(xls, p, de):
    ep = xls.shape[0]
    blk = 512
    return pl.pallas_call(
        _contrib_body,
        grid=(ep // blk,),
        in_specs=[
            pl.BlockSpec((blk, _H * _D), lambda i: (i, 0)),
            pl.BlockSpec((blk, _D), lambda i: (i, 0)),
            pl.BlockSpec((blk, _D), lambda i: (i, 0)),
        ],
        out_specs=pl.BlockSpec((blk, _D), lambda i: (i, 0)),
        out_shape=jax.ShapeDtypeStruct((ep, _D), jnp.float32),
    )(xls, p, de)


def _proj_body(z_ref, wl_ref, bl_ref, wr_ref, br_ref, xl_ref, xr_ref):
    z = z_ref[...]
    xl_ref[...] = jnp.dot(z, wl_ref[...], preferred_element_type=jnp.float32) + bl_ref[...]
    xr_ref[...] = jnp.dot(z, wr_ref[...], preferred_element_type=jnp.float32) + br_ref[...]


def _proj(z, WlT, bl, WrT, br):
    n, cin = z.shape
    cout = WlT.shape[1]
    blk = 400
    grid = n // blk
    return pl.pallas_call(
        _proj_body,
        grid=(grid,),
        in_specs=[
            pl.BlockSpec((blk, cin), lambda i: (i, 0)),
            pl.BlockSpec((cin, cout), lambda i: (0, 0)),
            pl.BlockSpec((1, cout), lambda i: (0, 0)),
            pl.BlockSpec((cin, cout), lambda i: (0, 0)),
            pl.BlockSpec((1, cout), lambda i: (0, 0)),
        ],
        out_specs=[
            pl.BlockSpec((blk, cout), lambda i: (i, 0)),
            pl.BlockSpec((blk, cout), lambda i: (i, 0)),
        ],
        out_shape=[jax.ShapeDtypeStruct((n, cout), jnp.float32)] * 2,
    )(z, WlT, bl.reshape(1, -1), WrT, br.reshape(1, -1))


def _post_body(y0_ref, y1_ref, bias_ref, bnw_ref, bnb_ref, res_ref, o_ref):
    y = y0_ref[...] + y1_ref[...] + bias_ref[...]
    m = jnp.mean(y, axis=0, keepdims=True)
    v = jnp.mean((y - m) * (y - m), axis=0, keepdims=True)
    z = (y - m) * jax.lax.rsqrt(v + 1e-5) * bnw_ref[...] + bnb_ref[...]
    z = jnp.maximum(z, 0.0)
    mm = jnp.mean(z, axis=-1, keepdims=True)
    vv = jnp.mean((z - mm) * (z - mm), axis=-1, keepdims=True)
    o_ref[...] = (z - mm) * jax.lax.rsqrt(vv + 1e-5) + res_ref[...]


def _post(y0, y1, bias, bnw, bnb, res):
    n, d = y0.shape
    return pl.pallas_call(
        _post_body,
        in_specs=[
            pl.BlockSpec((n, d), lambda: (0, 0)),
            pl.BlockSpec((n, d), lambda: (0, 0)),
            pl.BlockSpec((1, d), lambda: (0, 0)),
            pl.BlockSpec((1, d), lambda: (0, 0)),
            pl.BlockSpec((1, d), lambda: (0, 0)),
            pl.BlockSpec((n, d), lambda: (0, 0)),
        ],
        out_specs=pl.BlockSpec((n, d), lambda: (0, 0)),
        out_shape=jax.ShapeDtypeStruct((n, d), jnp.float32),
    )(y0, y1, bias.reshape(1, -1), bnw.reshape(1, -1), bnb.reshape(1, -1), res)


def _bn_rows(z, w, b):
    m = jnp.mean(z, axis=0, keepdims=True)
    v = jnp.mean((z - m) * (z - m), axis=0, keepdims=True)
    return (z - m) * jax.lax.rsqrt(v + 1e-5) * w + b


def _s2s_body(h_ref, bm_ref, wih_ref, whh_ref, bih_ref, bhh_ref,
              wrep_ref, brep_ref, bnwrep_ref, bnbrep_ref,
              h1_ref, h2_ref, o1_ref, o2_ref):
    h = h_ref[...]
    nb = o1_ref.shape[0]
    bcol = bm_ref[...]
    iot = jax.lax.broadcasted_iota(jnp.int32, (h.shape[0], nb), 1)
    M = (bcol == iot).astype(jnp.float32)

    wih = wih_ref[...]
    whh = whh_ref[...]
    bih = bih_ref[...]
    bhh = bhh_ref[...]

    d = h.shape[1]
    qs = jnp.zeros((nb, 2 * d), jnp.float32)
    hh = jnp.zeros((nb, d), jnp.float32)
    cc = jnp.zeros((nb, d), jnp.float32)
    for _ in range(3):
        gates = (jnp.dot(qs, wih, preferred_element_type=jnp.float32) + bih
                 + jnp.dot(hh, whh, preferred_element_type=jnp.float32) + bhh)
        gi = gates[:, 0 * d:1 * d]
        gf = gates[:, 1 * d:2 * d]
        gg = gates[:, 2 * d:3 * d]
        go = gates[:, 3 * d:4 * d]
        cc = jax.nn.sigmoid(gf) * cc + jax.nn.sigmoid(gi) * jnp.tanh(gg)
        hh = jax.nn.sigmoid(go) * jnp.tanh(cc)
        q = hh
        qb = jnp.dot(M, q, preferred_element_type=jnp.float32)
        e = jnp.sum(h * qb, axis=-1, keepdims=True)
        we = jnp.where(M > 0, e, -1e30)
        em = jnp.max(we, axis=0, keepdims=True)
        emn = jnp.dot(M, em.T, preferred_element_type=jnp.float32)
        ex = jnp.exp(e - emn)
        den = jax.lax.dot_general(M, ex, (((0,), (0,)), ((), ())),
                                  preferred_element_type=jnp.float32)
        a = ex / (jnp.dot(M, den, preferred_element_type=jnp.float32) + 1e-16)
        r = jax.lax.dot_general(M, a * h, (((0,), (0,)), ((), ())),
                                preferred_element_type=jnp.float32)
        qs = jnp.concatenate([q, r], axis=-1)

    rep = jnp.maximum(jnp.dot(qs, wrep_ref[...], preferred_element_type=jnp.float32)
                      + brep_ref[...], 0.0)
    rep = _bn_rows(rep, bnwrep_ref[...], bnbrep_ref[...])

    for hp_ref, o_ref in ((h1_ref, o1_ref), (h2_ref, o2_ref)):
        hp = hp_ref[...]
        w1 = hp[0:128, 0:64]
        b1 = hp[128:129, 0:64]
        bw1 = hp[129:130, 0:64]
        bb1 = hp[130:131, 0:64]
        w2 = hp[131:195, 0:32]
        b2 = hp[195:196, 0:32]
        bw2 = hp[196:197, 0:32]
        bb2 = hp[197:198, 0:32]
        w3 = hp[198:230, 0:1]
        b3 = hp[230:231, 0:1]
        z1 = jnp.maximum(jnp.dot(rep, w1, preferred_element_type=jnp.float32) + b1, 0.0)
        z1 = _bn_rows(z1, bw1, bb1)
        z2 = jnp.maximum(jnp.dot(z1, w2, preferred_element_type=jnp.float32) + b2, 0.0)
        z2 = _bn_rows(z2, bw2, bb2)
        o_ref[...] = jnp.dot(z2, w3, preferred_element_type=jnp.float32) + b3


def _pack_head(hp):
    p = jnp.zeros((232, 128), jnp.float32)
    p = p.at[0:128, 0:64].set(hp[0].T)
    p = p.at[128, 0:64].set(hp[1])
    p = p.at[129, 0:64].set(hp[2])
    p = p.at[130, 0:64].set(hp[3])
    p = p.at[131:195, 0:32].set(hp[4].T)
    p = p.at[195, 0:32].set(hp[5])
    p = p.at[196, 0:32].set(hp[6])
    p = p.at[197, 0:32].set(hp[7])
    p = p.at[198:230, 0:1].set(hp[8].T)
    p = p.at[230, 0:1].set(hp[9])
    return p


def _s2s(h, batch, lstm, rep, gcgr, glp1r):
    n, d = h.shape
    nb = 64
    Wih, Whh, bih, bhh = lstm
    Wr_, br_, bnw, bnb = rep
    args = (h, batch.astype(jnp.int32).reshape(n, 1),
            Wih.T, Whh.T, bih.reshape(1, -1), bhh.reshape(1, -1),
            Wr_.T, br_.reshape(1, -1), bnw.reshape(1, -1), bnb.reshape(1, -1),
            _pack_head(gcgr), _pack_head(glp1r))
    full = lambda a: pl.BlockSpec(a.shape, lambda: (0,) * a.ndim)
    return pl.pallas_call(
        _s2s_body,
        in_specs=[full(a) for a in args],
        out_specs=[pl.BlockSpec((nb, 1), lambda: (0, 0))] * 2,
        out_shape=[jax.ShapeDtypeStruct((nb, 1), jnp.float32)] * 2,
    )(*args)


def kernel(x, edge_index, batch, params):
    n = x.shape[0]
    ne = edge_index.shape[1] + n
    ep = ((ne + _NW * _C - 1) // (_NW * _C)) * (_NW * _C)
    loops = jnp.arange(n, dtype=jnp.int32)
    padi = jnp.full((ep - ne,), n, jnp.int32)
    src3 = jnp.concatenate([edge_index[0].astype(jnp.int32), loops, padi]
                           ).reshape(_NW, -1, _C)
    dst3 = jnp.concatenate([edge_index[1].astype(jnp.int32), loops, padi]
                           ).reshape(_NW, -1, _C)
    zer128 = jnp.zeros((8, _D), jnp.float32)
    padrows = jnp.zeros((_TP - n, _H * _D), jnp.float32)

    h = x
    res = None
    for i, cp in enumerate(params["convs"]):
        Wl, bl, Wr, br, att, bias, bnw, bnb = cp
        if i > 0:
            res = h
        xl, xr = _proj(h, Wl.T, bl, Wr.T, br)
        xlp = jnp.concatenate([xl, padrows])
        xrp = jnp.concatenate([xr, padrows])
        attp = jnp.zeros((8, _D), jnp.float32).at[:_H].set(att)
        xls = _sc_gather(xlp, src3)
        xrd = _sc_gather(xrp, dst3)
        p = _tc_logits(xls, xrd, attp)
        denp = _sc_scatter_add(p, dst3, zer128)
        den = _tc_add2(denp[0], denp[1])
        de = _sc_gather(den, dst3)
        w = _tc_contrib(xls, p, de)
        outp = _sc_scatter_add(w, dst3, zer128)
        rz = res if i == 1 else jnp.zeros((n, _D), jnp.float32)
        h = _post(outp[0, :n], outp[1, :n], bias, bnw, bnb, rz)

    return _s2s(h, batch, params["lstm"], params["rep"],
                params["gcgr"], params["glp1r"])
